# baseline (speedup 1.0000x reference)
import functools

import jax
import jax.numpy as jnp
from jax import lax
from jax.experimental import pallas as pl
from jax.experimental.pallas import tpu as pltpu
from jax.experimental.pallas import tpu_sc as plsc


@functools.cache
def _make_embed(V, D, N, P, scale):
    info = plsc.get_sparse_core_info()
    NC, NS, L = info.num_cores, info.num_subcores, info.num_lanes
    NW = NC * NS
    assert N % NW == 0 and D % L == 0 and V % 2 == 0
    NPW = N // NW
    W = 256
    H = NPW // W
    assert H == 2
    E = NPW * P
    assert E % 128 == 0 and W % L == 0
    SR = E // 128
    G = W // L
    mesh = plsc.VectorSubcoreMesh(core_axis_name="c", subcore_axis_name="s")

    @functools.partial(
        pl.kernel,
        mesh=mesh,
        compiler_params=pltpu.CompilerParams(needs_layout_passes=False),
        out_type=jax.ShapeDtypeStruct((P, D, N), jnp.float32),
        scratch_types=[
            pltpu.VMEM((SR, 128), jnp.int32),
            pltpu.VMEM((W,), jnp.int32),
            pltpu.VMEM((W,), jnp.int32),
            pltpu.VMEM((W,), jnp.int32),
            pltpu.VMEM((W,), jnp.int32),
            pltpu.VMEM((W, 2 * D), jnp.float32),
            pltpu.VMEM((W, 2 * D), jnp.float32),
            pltpu.VMEM((D, W), jnp.float32),
            pltpu.VMEM((D, W), jnp.float32),
            pltpu.SemaphoreType.DMA,
            pltpu.SemaphoreType.DMA,
            pltpu.SemaphoreType.DMA,
            pltpu.SemaphoreType.DMA,
        ],
    )
    def k(idx_hbm, table_hbm, out_hbm, slab, gidx0, gidx1, colb0, colb1,
          rows0, rows1, outp0, outp1, gsem0, gsem1, wsem0, wsem1):
        wid = lax.axis_index("s") * NC + lax.axis_index("c")
        iota = lax.iota(jnp.int32, L)
        obase = wid * NPW
        pltpu.sync_copy(idx_hbm.at[wid], slab)
        gidx = (gidx0, gidx1)
        colb = (colb0, colb1)
        rows = (rows0, rows1)
        outp = (outp0, outp1)
        gsem = (gsem0, gsem1)
        wsem = (wsem0, wsem1)

        def build(p, half, buf):
            @plsc.parallel_loop(0, G, unroll=4)
            def _(g):
                e = (half * W + g * L + iota) * P + p
                v = plsc.load_gather(slab, [lax.shift_right_logical(e, 7),
                                            lax.bitwise_and(e, 127)])
                sl = pl.ds(g * L, L)
                gidx[buf][sl] = lax.shift_right_logical(v, 1)
                colb[buf][sl] = lax.bitwise_and(v, 1) * D

        def fire(buf):
            pltpu.async_copy(table_hbm.at[gidx[buf]], rows[buf], gsem[buf])

        def gwait(buf):
            pltpu.make_async_copy(table_hbm.at[gidx[buf]], rows[buf],
                                  gsem[buf]).wait()

        def trans(buf):
            cbs = [colb[buf][pl.ds(g * L, L)] for g in range(G)]
            rvs = [g * L + iota for g in range(G)]

            @plsc.parallel_loop(0, D, unroll=8)
            def _(d):
                for g in range(G):
                    x = plsc.load_gather(rows[buf], [rvs[g], cbs[g] + d])
                    outp[buf][d, pl.ds(g * L, L)] = x * scale

        def wfire(p, half, buf):
            pltpu.async_copy(outp[buf],
                             out_hbm.at[p, :, pl.ds(obase + half * W, W)],
                             wsem[buf])

        def wwait(buf):
            pltpu.make_async_copy(outp[buf],
                                  out_hbm.at[0, :, pl.ds(obase, W)],
                                  wsem[buf]).wait()

        build(0, 0, 0)
        fire(0)

        def plane_body(p, carry):
            build(p, 1, 1)
            fire(1)
            gwait(0)

            @pl.when(p > 0)
            def _():
                wwait(0)

            trans(0)
            wfire(p, 0, 0)

            @pl.when(p < P - 1)
            def _():
                build(p + 1, 0, 0)
                fire(0)

            gwait(1)

            @pl.when(p > 0)
            def _():
                wwait(1)

            trans(1)
            wfire(p, 1, 1)
            return carry

        lax.fori_loop(0, P, plane_body, 0, unroll=False)
        wwait(0)
        wwait(1)

    return k


def kernel(input, weight):
    V, D = weight.shape
    N, P = input.shape
    NW = 32
    scale = float(D) ** 0.5
    idx3 = input.reshape(NW, (N // NW) * P // 128, 128).astype(jnp.int32)
    table2 = jnp.concatenate([weight[0::2, :], weight[1::2, :]], axis=1)
    out3 = _make_embed(V, D, N, P, scale)(idx3, table2)
    return jnp.transpose(out3, (2, 0, 1))


# speedup vs baseline: 7.5097x; 7.5097x over previous
import functools

import jax
import jax.numpy as jnp
from jax import lax
from jax.experimental import pallas as pl
from jax.experimental.pallas import tpu as pltpu
from jax.experimental.pallas import tpu_sc as plsc


@functools.cache
def _make_embed(V, D, N, P, scale):
    info = plsc.get_sparse_core_info()
    NC, NS, L = info.num_cores, info.num_subcores, info.num_lanes
    NW = NC * NS
    assert N % NW == 0 and D % L == 0 and V % 2 == 0
    NPW = N // NW
    W = 256
    H = NPW // W
    assert H == 2
    E = NPW * P
    assert E % 128 == 0 and W % L == 0
    SR = E // 128
    G = W // L
    mesh = plsc.VectorSubcoreMesh(core_axis_name="c", subcore_axis_name="s")

    @functools.partial(
        pl.kernel,
        mesh=mesh,
        compiler_params=pltpu.CompilerParams(needs_layout_passes=False),
        out_type=jax.ShapeDtypeStruct((P, D, N), jnp.float32),
        scratch_types=[
            pltpu.VMEM((SR, 128), jnp.int32),
            pltpu.VMEM((W,), jnp.int32),
            pltpu.VMEM((W,), jnp.int32),
            pltpu.VMEM((W,), jnp.int32),
            pltpu.VMEM((W,), jnp.int32),
            pltpu.VMEM((W, 2 * D), jnp.float32),
            pltpu.VMEM((W, 2 * D), jnp.float32),
            pltpu.VMEM((D, W), jnp.float32),
            pltpu.VMEM((D, W), jnp.float32),
            pltpu.SemaphoreType.DMA,
            pltpu.SemaphoreType.DMA,
            pltpu.SemaphoreType.DMA,
            pltpu.SemaphoreType.DMA,
        ],
    )
    def k(idx_hbm, table_hbm, out_hbm, slab, gidx0, gidx1, colb0, colb1,
          rows0, rows1, outp0, outp1, gsem0, gsem1, wsem0, wsem1):
        wid = lax.axis_index("s") * NC + lax.axis_index("c")
        iota = lax.iota(jnp.int32, L)
        obase = wid * NPW
        pltpu.sync_copy(idx_hbm.at[wid], slab)
        gidx = (gidx0, gidx1)
        colb = (colb0, colb1)
        rows = (rows0, rows1)
        outp = (outp0, outp1)
        gsem = (gsem0, gsem1)
        wsem = (wsem0, wsem1)

        def build(p, half, buf):
            @plsc.parallel_loop(0, G, unroll=4)
            def _(g):
                e = (half * W + g * L + iota) * P + p
                v = plsc.load_gather(slab, [lax.shift_right_logical(e, 7),
                                            lax.bitwise_and(e, 127)])
                sl = pl.ds(g * L, L)
                gidx[buf][sl] = lax.shift_right_logical(v, 1)
                colb[buf][sl] = lax.bitwise_and(v, 1) * D

        def fire(buf):
            pltpu.async_copy(table_hbm.at[gidx[buf]], rows[buf], gsem[buf])

        def gwait(buf):
            pltpu.make_async_copy(table_hbm.at[gidx[buf]], rows[buf],
                                  gsem[buf]).wait()

        def trans(buf):
            cbs = [colb[buf][pl.ds(g * L, L)] for g in range(G)]
            rvs = [g * L + iota for g in range(G)]

            @plsc.parallel_loop(0, D, unroll=8)
            def _(d):
                for g in range(G):
                    x = plsc.load_gather(rows[buf], [rvs[g], cbs[g] + d])
                    outp[buf][d, pl.ds(g * L, L)] = x * scale

        def wfire(p, half, buf):
            pltpu.async_copy(outp[buf],
                             out_hbm.at[p, :, pl.ds(obase + half * W, W)],
                             wsem[buf])

        def wwait(buf):
            pltpu.make_async_copy(outp[buf],
                                  out_hbm.at[0, :, pl.ds(obase, W)],
                                  wsem[buf]).wait()

        build(0, 0, 0)
        fire(0)

        def plane_body(p, carry):
            build(p, 1, 1)
            fire(1)
            gwait(0)

            @pl.when(p > 0)
            def _():
                wwait(0)

            trans(0)
            wfire(p, 0, 0)

            @pl.when(p < P - 1)
            def _():
                build(p + 1, 0, 0)
                fire(0)

            gwait(1)

            @pl.when(p > 0)
            def _():
                wwait(1)

            trans(1)
            wfire(p, 1, 1)
            return carry

        lax.fori_loop(0, P, plane_body, 0, unroll=False)
        wwait(0)
        wwait(1)

    return k


def kernel(input, weight):
    V, D = weight.shape
    N, P = input.shape
    NW = 32
    scale = float(D) ** 0.5
    idx3 = input.reshape(NW, (N // NW) * P // 128, 128).astype(jnp.int32)
    table2 = weight.reshape(V // 2, 2 * D)
    out3 = _make_embed(V, D, N, P, scale)(idx3, table2)
    return jnp.transpose(out3, (2, 0, 1))
